# baseline (speedup 1.0000x reference)
import functools

import jax
import jax.numpy as jnp
from jax import lax
from jax.experimental import pallas as pl
from jax.experimental.pallas import tpu as pltpu
from jax.experimental.pallas import tpu_sc as plsc

_C = 32
_DZ = 6
_ZTC = 72
_ZSC = 96 - _ZTC
_NW = 32
_NU = 2 * _ZSC * 2


def _part_kernel(net_ref, tgt_ref, stu_ref, tea_ref, sums_ref, cnts_ref):
    t = pl.program_id(1)
    n0 = net_ref[0, 0:1]
    n1 = net_ref[0, 1:2]
    n2 = net_ref[0, 2:3]
    tgt = tgt_ref[0, 0:1]
    pred0 = (n0 >= n1) & (n0 >= n2)
    pred1 = (~pred0) & (n1 >= n2)
    kid = (tgt == 1) & pred0
    is2 = tgt == 2
    tum = is2 & pred1
    wrong = is2 & (~pred1)
    kidf = kid.astype(jnp.float32).reshape(1, _DZ, 12, 8, 96)
    tumf = tum.astype(jnp.float32).reshape(1, _DZ, 12, 8, 96)
    wrongf = wrong.astype(jnp.float32).reshape(1, _DZ, 12, 8, 96)
    stu = stu_ref[0].reshape(_C, _DZ, 12, 8, 96)
    tea = tea_ref[0].reshape(_C, _DZ, 12, 8, 96)
    kid_part = jnp.sum(stu * kidf, axis=(1, 2))
    wrong_part = jnp.sum(stu * wrongf, axis=(1, 2))
    tum_part = jnp.sum(tea * tumf, axis=(1, 2))
    sums = jnp.stack([kid_part, tum_part, wrong_part], axis=0)
    cnts = jnp.concatenate(
        [jnp.sum(kidf, axis=(1, 2)), jnp.sum(tumf, axis=(1, 2)),
         jnp.sum(wrongf, axis=(1, 2))],
        axis=0,
    )

    @pl.when(t == 0)
    def _():
        sums_ref[0] = sums
        cnts_ref[0] = cnts

    @pl.when(t != 0)
    def _():
        sums_ref[0] += sums
        cnts_ref[0] += cnts


def _sc_unit(u, net_hbm, tgt_hbm, stu_hbm, tea_hbm, out_hbm,
             n0_v, n1_v, n2_v, tg_v, km_v, sp_v, tp_v, acc_v,
             sem_m, sem_s0, sem_t0, sem_s1, sem_t1):
    half = u % 2
    z_rel = (u // 2) % _ZSC
    b = u // (2 * _ZSC)
    z = _ZTC + z_rel
    pltpu.make_async_copy(net_hbm.at[b, 0, z], n0_v, sem_m).start()
    pltpu.make_async_copy(net_hbm.at[b, 1, z], n1_v, sem_m).start()
    pltpu.make_async_copy(net_hbm.at[b, 2, z], n2_v, sem_m).start()
    pltpu.make_async_copy(tgt_hbm.at[b, 0, z], tg_v, sem_m).start()
    pltpu.make_async_copy(net_hbm.at[b, 0, z], n0_v, sem_m).wait()
    pltpu.make_async_copy(net_hbm.at[b, 1, z], n1_v, sem_m).wait()
    pltpu.make_async_copy(net_hbm.at[b, 2, z], n2_v, sem_m).wait()
    pltpu.make_async_copy(tgt_hbm.at[b, 0, z], tg_v, sem_m).wait()

    def mask_row(r, carry):
        ck2, ct2, cw2 = carry
        for c in range(6):
            sl = pl.ds(c * 16, 16)
            a0 = n0_v[r, sl]
            a1 = n1_v[r, sl]
            a2 = n2_v[r, sl]
            tg = tg_v[r, sl]
            ge01 = jnp.where(a0 >= a1, 1.0, 0.0)
            ge02 = jnp.where(a0 >= a2, 1.0, 0.0)
            ge12 = jnp.where(a1 >= a2, 1.0, 0.0)
            p0f = ge01 * ge02
            p1f = (1.0 - p0f) * ge12
            t1f = jnp.where(tg == 1, 1.0, 0.0)
            t2f = jnp.where(tg == 2, 1.0, 0.0)
            kid = t1f * p0f
            tum = t2f * p1f
            wrong = t2f - tum
            km_v[r, sl] = kid + 2.0 * tum + 4.0 * wrong
            ck2, ct2, cw2 = ck2 + kid, ct2 + tum, cw2 + wrong
        return ck2, ct2, cw2

    zero = jnp.zeros((16,), jnp.float32)
    ck, ct, cw = lax.fori_loop(0, 96, mask_row, (zero, zero, zero))
    acc_v[0, 16] = ck
    acc_v[1, 16] = ct
    acc_v[2, 16] = cw

    ch0 = half * 16
    sems = ((sem_s0, sem_t0), (sem_s1, sem_t1))

    def start_pair(c, buf):
        ss, st = sems[buf]
        pltpu.make_async_copy(stu_hbm.at[b, ch0 + c, z], sp_v.at[buf], ss).start()
        pltpu.make_async_copy(tea_hbm.at[b, ch0 + c, z], tp_v.at[buf], st).start()

    def wait_pair(c, buf):
        ss, st = sems[buf]
        pltpu.make_async_copy(stu_hbm.at[b, ch0 + c, z], sp_v.at[buf], ss).wait()
        pltpu.make_async_copy(tea_hbm.at[b, ch0 + c, z], tp_v.at[buf], st).wait()

    start_pair(0, 0)
    for c in range(16):
        buf = c % 2
        if c < 15:
            start_pair(c + 1, 1 - buf)
        wait_pair(c, buf)

        def feat_row(r, carry):
            ak2, at2, aw2 = carry
            for cc in range(6):
                sl = pl.ds(cc * 16, 16)
                s = sp_v[buf, r, sl]
                te = tp_v[buf, r, sl]
                m = km_v[r, sl]
                ak2 = ak2 + s * jnp.where(m == 1.0, 1.0, 0.0)
                at2 = at2 + te * jnp.where(m == 2.0, 1.0, 0.0)
                aw2 = aw2 + s * jnp.where(m == 4.0, 1.0, 0.0)
            return ak2, at2, aw2

        ak, at_, aw = lax.fori_loop(0, 96, feat_row, (zero, zero, zero))
        acc_v[0, c] = ak
        acc_v[1, c] = at_
        acc_v[2, c] = aw

    pltpu.sync_copy(acc_v, out_hbm.at[u])


def _sc_kernel(net_hbm, tgt_hbm, stu_hbm, tea_hbm, out_hbm,
               n0_v, n1_v, n2_v, tg_v, km_v, sp_v, tp_v, acc_v,
               sem_m, sem_s0, sem_t0, sem_s1, sem_t1):
    wid = lax.axis_index("s") * 2 + lax.axis_index("c")
    for rep in range(_NU // _NW):
        _sc_unit(rep * _NW + wid, net_hbm, tgt_hbm, stu_hbm, tea_hbm,
                 out_hbm, n0_v, n1_v, n2_v, tg_v, km_v,
                 sp_v, tp_v, acc_v, sem_m, sem_s0, sem_t0, sem_s1, sem_t1)


def _norm(v):
    return v / (jnp.sqrt(jnp.sum(v * v, axis=-1, keepdims=True)) + 1e-8)


def _loss_kernel(spatial, nb, nd, sums_ref, cnts_ref, sc_ref, deque_ref,
                 out_ref):
    sums = sums_ref[:]
    cnts = jnp.sum(cnts_ref[:], axis=(-2, -1))
    sc = jnp.sum(sc_ref[:], axis=-1)
    sc_ch = jnp.concatenate([sc[:, 0, :, 0:16], sc[:, 1, :, 0:16]],
                            axis=-1)
    sc_cnt = sc[:, 0, :, 16] + sc[:, 1, :, 16]
    cnts = cnts + sc_cnt
    vecs = (jnp.sum(sums, axis=(-2, -1)) + sc_ch) / spatial
    kid_n = _norm(vecs[:, 0, :])
    tum_n = _norm(vecs[:, 1, :])
    tgt_n = _norm(vecs[:, 2, :])
    dq_n = _norm(deque_ref[:])
    ext = jnp.concatenate([dq_n, kid_n], axis=0)
    kid_sim = jax.lax.dot_general(
        tgt_n, ext, (((1,), (1,)), ((), ())),
        preferred_element_type=jnp.float32)
    tum_sim = jnp.sum(tgt_n * tum_n, axis=-1, keepdims=True)
    active_f = ((cnts[:, 1:2] != 0).astype(jnp.float32)
                * (cnts[:, 2:3] != 0).astype(jnp.float32))
    iext = jax.lax.broadcasted_iota(jnp.int32, (nb, nd + nb), 0)
    jext = jax.lax.broadcasted_iota(jnp.int32, (nb, nd + nb), 1)
    valid_f = ((jext - nd) <= iext).astype(jnp.float32)
    for j in range(nb):
        kvf = jnp.where(cnts[j, 0] != 0, 1.0, 0.0)
        valid_f = valid_f * jnp.where(jext == nd + j, kvf, 1.0)
    exp_t = active_f * jnp.exp(tum_sim)
    exp_k = active_f * valid_f * jnp.exp(kid_sim)
    check = jnp.sum(active_f) > 0.0
    loss = jnp.where(
        check,
        (-1.0 / nb) * jnp.log(jnp.sum(exp_t) / jnp.sum(exp_k)),
        0.0,
    )
    out_ref[:, :] = jnp.full((1, 1), loss, jnp.float32)


def kernel(net_output, student_feature, teacher_feature, target, kidney_deque):
    B = net_output.shape[0]
    spatial = net_output.shape[2] * net_output.shape[3] * net_output.shape[4]
    D = kidney_deque.shape[0]
    nt = _ZTC // _DZ
    sums, cnts = pl.pallas_call(
        _part_kernel,
        grid=(B, nt),
        in_specs=[
            pl.BlockSpec((1, 3, _DZ, 96, 96), lambda b, t: (b, 0, t, 0, 0)),
            pl.BlockSpec((1, 1, _DZ, 96, 96), lambda b, t: (b, 0, t, 0, 0)),
            pl.BlockSpec((1, _C, _DZ, 96, 96), lambda b, t: (b, 0, t, 0, 0)),
            pl.BlockSpec((1, _C, _DZ, 96, 96), lambda b, t: (b, 0, t, 0, 0)),
        ],
        out_specs=[
            pl.BlockSpec((1, 3, _C, 8, 96), lambda b, t: (b, 0, 0, 0, 0)),
            pl.BlockSpec((1, 3, 8, 96), lambda b, t: (b, 0, 0, 0)),
        ],
        out_shape=[
            jax.ShapeDtypeStruct((B, 3, _C, 8, 96), jnp.float32),
            jax.ShapeDtypeStruct((B, 3, 8, 96), jnp.float32),
        ],
        compiler_params=pltpu.CompilerParams(
            dimension_semantics=("parallel", "arbitrary")),
    )(net_output, target, student_feature, teacher_feature)
    sc_mesh = plsc.VectorSubcoreMesh(core_axis_name="c", subcore_axis_name="s")
    sc_out = pl.kernel(
        _sc_kernel,
        mesh=sc_mesh,
        out_type=jax.ShapeDtypeStruct((_NU, 3, 17, 16), jnp.float32),
        scratch_types=[
            pltpu.VMEM((96, 96), jnp.float32),
            pltpu.VMEM((96, 96), jnp.float32),
            pltpu.VMEM((96, 96), jnp.float32),
            pltpu.VMEM((96, 96), jnp.int32),
            pltpu.VMEM((96, 96), jnp.float32),
            pltpu.VMEM((2, 96, 96), jnp.float32),
            pltpu.VMEM((2, 96, 96), jnp.float32),
            pltpu.VMEM((3, 17, 16), jnp.float32),
            pltpu.SemaphoreType.DMA,
            pltpu.SemaphoreType.DMA,
            pltpu.SemaphoreType.DMA,
            pltpu.SemaphoreType.DMA,
            pltpu.SemaphoreType.DMA,
        ],
    )(net_output, target, student_feature, teacher_feature)
    sc_m = sc_out.reshape(B, _ZSC, 2, 3, 17, 16).sum(axis=1)
    loss = pl.pallas_call(
        functools.partial(_loss_kernel, float(spatial), B, D),
        out_shape=jax.ShapeDtypeStruct((1, 1), jnp.float32),
    )(sums, cnts, sc_m, kidney_deque)
    return loss[0, 0]


# speedup vs baseline: 1.1301x; 1.1301x over previous
import functools

import jax
import jax.numpy as jnp
from jax.experimental import pallas as pl
from jax.experimental.pallas import tpu as pltpu

_C = 32
_DZ = 8


def _part_kernel(net_ref, tgt_ref, stu_ref, tea_ref, sums_ref, cnts_ref):
    t = pl.program_id(1)
    n0 = net_ref[0, 0:1]
    n1 = net_ref[0, 1:2]
    n2 = net_ref[0, 2:3]
    tgt = tgt_ref[0, 0:1]
    pred0 = (n0 >= n1) & (n0 >= n2)
    pred1 = (~pred0) & (n1 >= n2)
    kid = (tgt == 1) & pred0
    is2 = tgt == 2
    tum = is2 & pred1
    wrong = is2 & (~pred1)
    kidf = kid.astype(jnp.float32).reshape(1, _DZ, 12, 8, 96)
    tumf = tum.astype(jnp.float32).reshape(1, _DZ, 12, 8, 96)
    wrongf = wrong.astype(jnp.float32).reshape(1, _DZ, 12, 8, 96)
    stu = stu_ref[0].reshape(_C, _DZ, 12, 8, 96)
    tea = tea_ref[0].reshape(_C, _DZ, 12, 8, 96)
    kid_part = wrong_part = tum_part = None
    for z in range(_DZ):
        kp = jnp.sum(stu[:, z] * kidf[:, z], axis=1)
        wp = jnp.sum(stu[:, z] * wrongf[:, z], axis=1)
        tp = jnp.sum(tea[:, z] * tumf[:, z], axis=1)
        if z == 0:
            kid_part, wrong_part, tum_part = kp, wp, tp
        else:
            kid_part = kid_part + kp
            wrong_part = wrong_part + wp
            tum_part = tum_part + tp
    sums = jnp.stack([kid_part, tum_part, wrong_part], axis=0)
    cnts = jnp.concatenate(
        [jnp.sum(kidf, axis=(1, 2)), jnp.sum(tumf, axis=(1, 2)),
         jnp.sum(wrongf, axis=(1, 2))],
        axis=0,
    )

    @pl.when(t == 0)
    def _():
        sums_ref[0] = sums
        cnts_ref[0] = cnts

    @pl.when(t != 0)
    def _():
        sums_ref[0] += sums
        cnts_ref[0] += cnts


def _norm(v):
    return v / (jnp.sqrt(jnp.sum(v * v, axis=-1, keepdims=True)) + 1e-8)


def _loss_kernel(spatial, nb, nd, sums_ref, cnts_ref, deque_ref, out_ref):
    sums = sums_ref[:]
    cnts = jnp.sum(cnts_ref[:], axis=(-2, -1))
    vecs = jnp.sum(sums, axis=(-2, -1)) / spatial
    kid_n = _norm(vecs[:, 0, :])
    tum_n = _norm(vecs[:, 1, :])
    tgt_n = _norm(vecs[:, 2, :])
    dq_n = _norm(deque_ref[:])
    ext = jnp.concatenate([dq_n, kid_n], axis=0)
    kid_sim = jax.lax.dot_general(
        tgt_n, ext, (((1,), (1,)), ((), ())),
        preferred_element_type=jnp.float32)
    tum_sim = jnp.sum(tgt_n * tum_n, axis=-1, keepdims=True)
    active_f = ((cnts[:, 1:2] != 0).astype(jnp.float32)
                * (cnts[:, 2:3] != 0).astype(jnp.float32))
    iext = jax.lax.broadcasted_iota(jnp.int32, (nb, nd + nb), 0)
    jext = jax.lax.broadcasted_iota(jnp.int32, (nb, nd + nb), 1)
    valid_f = ((jext - nd) <= iext).astype(jnp.float32)
    for j in range(nb):
        kvf = jnp.where(cnts[j, 0] != 0, 1.0, 0.0)
        valid_f = valid_f * jnp.where(jext == nd + j, kvf, 1.0)
    exp_t = active_f * jnp.exp(tum_sim)
    exp_k = active_f * valid_f * jnp.exp(kid_sim)
    check = jnp.sum(active_f) > 0.0
    loss = jnp.where(
        check,
        (-1.0 / nb) * jnp.log(jnp.sum(exp_t) / jnp.sum(exp_k)),
        0.0,
    )
    out_ref[:, :] = jnp.full((1, 1), loss, jnp.float32)


def kernel(net_output, student_feature, teacher_feature, target, kidney_deque):
    B = net_output.shape[0]
    nz = net_output.shape[2]
    spatial = net_output.shape[2] * net_output.shape[3] * net_output.shape[4]
    D = kidney_deque.shape[0]
    nt = nz // _DZ
    sums, cnts = pl.pallas_call(
        _part_kernel,
        grid=(B, nt),
        in_specs=[
            pl.BlockSpec((1, 3, _DZ, 96, 96), lambda b, t: (b, 0, t, 0, 0)),
            pl.BlockSpec((1, 1, _DZ, 96, 96), lambda b, t: (b, 0, t, 0, 0)),
            pl.BlockSpec((1, _C, _DZ, 96, 96), lambda b, t: (b, 0, t, 0, 0)),
            pl.BlockSpec((1, _C, _DZ, 96, 96), lambda b, t: (b, 0, t, 0, 0)),
        ],
        out_specs=[
            pl.BlockSpec((1, 3, _C, 8, 96), lambda b, t: (b, 0, 0, 0, 0)),
            pl.BlockSpec((1, 3, 8, 96), lambda b, t: (b, 0, 0, 0)),
        ],
        out_shape=[
            jax.ShapeDtypeStruct((B, 3, _C, 8, 96), jnp.float32),
            jax.ShapeDtypeStruct((B, 3, 8, 96), jnp.float32),
        ],
        compiler_params=pltpu.CompilerParams(
            dimension_semantics=("parallel", "arbitrary")),
    )(net_output, target, student_feature, teacher_feature)
    loss = pl.pallas_call(
        functools.partial(_loss_kernel, float(spatial), B, D),
        out_shape=jax.ShapeDtypeStruct((1, 1), jnp.float32),
    )(sums, cnts, kidney_deque)
    return loss[0, 0]
